# baseline (speedup 1.0000x reference)
import functools

import jax
import jax.numpy as jnp
from jax import lax
from jax.experimental import pallas as pl
from jax.experimental.pallas import tpu as pltpu
from jax.experimental.pallas import tpu_sc as plsc

N = 10000
E = 320000
D = 128
DE = 16

N_PAD = 10240
NW = 32
E_PER_W = E // NW
CHUNK = 64
EPW_PAD = 10240
N_CHUNKS = EPW_PAD // CHUNK
E_IDX_PAD = NW * EPW_PAD
E_ATTR_PAD = 327680


def _edge_mlp_body(ea_ref, we_ref, be_ref, out_ref):
    out_ref[...] = (
        jnp.dot(ea_ref[...], we_ref[...], preferred_element_type=jnp.float32)
        + be_ref[...]
    )


def _edge_mlp(edge_attr_pad, We, be):
    blk = 8192
    grid = E_ATTR_PAD // blk
    return pl.pallas_call(
        _edge_mlp_body,
        grid=(grid,),
        in_specs=[
            pl.BlockSpec((blk, DE), lambda i: (i, 0)),
            pl.BlockSpec((DE, D), lambda i: (0, 0)),
            pl.BlockSpec((1, D), lambda i: (0, 0)),
        ],
        out_specs=pl.BlockSpec((blk, D), lambda i: (i, 0)),
        out_shape=jax.ShapeDtypeStruct((E_ATTR_PAD, D), jnp.float32),
    )(edge_attr_pad, We, be.reshape(1, D))


def _sc_msgpass_body(x_hbm, e_hbm, src_hbm, dst_hbm, out_hbm,
                     sv0, sv1, dv0, dv1, xg0, xg1, ev0, ev1, zbuf, aggr,
                     isem0, isem1, gsem0, gsem1, esem0, esem1):
    c = lax.axis_index("c")
    s = lax.axis_index("s")
    wid = s * 2 + c
    ibase = wid * EPW_PAD
    ebase = wid * E_PER_W

    xg = (xg0, xg1)
    ev = (ev0, ev1)
    sv = (sv0, sv1)
    dv = (dv0, dv1)
    isem = (isem0, isem1)
    gsem = (gsem0, gsem1)
    esem = (esem0, esem1)

    def zrow(i, _):
        for j in range(D // 16):
            zbuf[i, pl.ds(j * 16, 16)] = jnp.zeros((16,), jnp.float32)
        return 0
    lax.fori_loop(0, 16, zrow, 0)

    rows_per_sub = N_PAD // 16
    def zcp(t, _):
        pltpu.sync_copy(zbuf, aggr.at[pl.ds(s * rows_per_sub + t * 16, 16)])
        return 0
    lax.fori_loop(0, rows_per_sub // 16, zcp, 0)

    plsc.subcore_barrier()

    def issue_idx(g, slot):
        pltpu.async_copy(src_hbm.at[pl.ds(ibase + g * CHUNK, CHUNK)],
                         sv[slot], isem[slot])
        pltpu.async_copy(dst_hbm.at[pl.ds(ibase + g * CHUNK, CHUNK)],
                         dv[slot], isem[slot])

    def wait_idx(slot):
        pltpu.make_async_copy(src_hbm.at[pl.ds(0, CHUNK)], sv[slot],
                              isem[slot]).wait()
        pltpu.make_async_copy(dst_hbm.at[pl.ds(0, CHUNK)], dv[slot],
                              isem[slot]).wait()

    def issue_data(g, slot):
        pltpu.async_copy(x_hbm.at[sv[slot]], xg[slot], gsem[slot])
        pltpu.async_copy(e_hbm.at[pl.ds(ebase + g * CHUNK, CHUNK)],
                         ev[slot], esem[slot])

    def wait_data(slot):
        pltpu.make_async_copy(x_hbm.at[sv[slot]], xg[slot], gsem[slot]).wait()
        pltpu.make_async_copy(e_hbm.at[pl.ds(0, CHUNK)], ev[slot],
                              esem[slot]).wait()

    def compute_scatter(slot):
        xgs, evs = xg[slot], ev[slot]
        with jax.named_scope("mp_comp"):
            def row(i, _):
                for j in range(D // 16):
                    sl = pl.ds(j * 16, 16)
                    xgs[i, sl] = jnp.maximum(xgs[i, sl] + evs[i, sl], 0.0)
                return 0
            lax.fori_loop(0, CHUNK, row, 0)
        with jax.named_scope("mp_scat"):
            pltpu.sync_copy(xgs, aggr.at[dv[slot]], add=True)

    def half(g, p, q):
        with jax.named_scope("mp_wait"):
            wait_data(p)
            wait_idx(q)
        issue_data(g + 1, q)
        compute_scatter(p)
        issue_idx(g + 2, p)

    issue_idx(0, 0)
    wait_idx(0)
    issue_data(0, 0)
    issue_idx(1, 1)

    def pair(t, _):
        half(2 * t, 0, 1)
        half(2 * t + 1, 1, 0)
        return 0
    lax.fori_loop(0, N_CHUNKS // 2 - 1, pair, 0)
    wait_data(0)
    wait_idx(1)
    issue_data(N_CHUNKS - 1, 1)
    compute_scatter(0)
    wait_data(1)
    compute_scatter(1)

    plsc.subcore_barrier()

    def cpout(t, _):
        r = s * rows_per_sub + t * 64
        pltpu.sync_copy(aggr.at[pl.ds(r, 64)], out_hbm.at[c, pl.ds(r, 64)])
        return 0
    lax.fori_loop(0, rows_per_sub // 64, cpout, 0)


def _sc_msgpass(x, e, src_pad, dst_pad):
    mesh = plsc.VectorSubcoreMesh(core_axis_name="c", subcore_axis_name="s")
    f = pl.kernel(
        _sc_msgpass_body,
        out_type=jax.ShapeDtypeStruct((2, N_PAD, D), jnp.float32),
        mesh=mesh,
        scratch_types=[
            pltpu.VMEM((CHUNK,), jnp.int32),
            pltpu.VMEM((CHUNK,), jnp.int32),
            pltpu.VMEM((CHUNK,), jnp.int32),
            pltpu.VMEM((CHUNK,), jnp.int32),
            pltpu.VMEM((CHUNK, D), jnp.float32),
            pltpu.VMEM((CHUNK, D), jnp.float32),
            pltpu.VMEM((CHUNK, D), jnp.float32),
            pltpu.VMEM((CHUNK, D), jnp.float32),
            pltpu.VMEM((16, D), jnp.float32),
            pltpu.VMEM_SHARED((N_PAD, D), jnp.float32),
            pltpu.SemaphoreType.DMA,
            pltpu.SemaphoreType.DMA,
            pltpu.SemaphoreType.DMA,
            pltpu.SemaphoreType.DMA,
            pltpu.SemaphoreType.DMA,
            pltpu.SemaphoreType.DMA,
        ],
    )
    return f(x, e, src_pad, dst_pad)


def _node_mlp_body(eps_ref, x_ref, p_ref, w1_ref, b1_ref, w2_ref, b2_ref,
                   out_ref):
    p = p_ref[0] + p_ref[1]
    h = (1.0 + eps_ref[0, 0]) * x_ref[...] + p
    t = jnp.dot(h, w1_ref[...], preferred_element_type=jnp.float32) + b1_ref[...]
    t = 0.5 * t * (1.0 + lax.erf(t * 0.7071067811865476))
    out_ref[...] = (
        jnp.dot(t, w2_ref[...], preferred_element_type=jnp.float32) + b2_ref[...]
    )


def _node_mlp(eps, x, partial, W1, b1, W2, b2):
    blk = 2000
    grid = N // blk
    return pl.pallas_call(
        _node_mlp_body,
        grid=(grid,),
        in_specs=[
            pl.BlockSpec(memory_space=pltpu.SMEM),
            pl.BlockSpec((blk, D), lambda i: (i, 0)),
            pl.BlockSpec((2, blk, D), lambda i: (0, i, 0)),
            pl.BlockSpec((D, D), lambda i: (0, 0)),
            pl.BlockSpec((1, D), lambda i: (0, 0)),
            pl.BlockSpec((D, D), lambda i: (0, 0)),
            pl.BlockSpec((1, D), lambda i: (0, 0)),
        ],
        out_specs=pl.BlockSpec((blk, D), lambda i: (i, 0)),
        out_shape=jax.ShapeDtypeStruct((N, D), jnp.float32),
    )(eps.reshape(1, 1), x, partial, W1, b1.reshape(1, D), W2, b2.reshape(1, D))


def kernel(x, edge_attr, edge_idx, eps0, We0, be0, W10, b10, W20, b20,
           eps1, We1, be1, W11, b11, W21, b21):
    pad_w = EPW_PAD - E_PER_W
    src = jnp.pad(edge_idx[0].reshape(NW, E_PER_W), ((0, 0), (0, pad_w)),
                  constant_values=0).reshape(E_IDX_PAD)
    dst = jnp.pad(edge_idx[1].reshape(NW, E_PER_W), ((0, 0), (0, pad_w)),
                  constant_values=N).reshape(E_IDX_PAD)
    ea_pad = jnp.pad(edge_attr, ((0, E_ATTR_PAD - E), (0, 0)))

    e0 = _edge_mlp(ea_pad, We0, be0)
    e1 = _edge_mlp(ea_pad, We1, be1)

    p0 = _sc_msgpass(x, e0, src, dst)
    h = _node_mlp(eps0, x, p0, W10, b10, W20, b20)

    p1 = _sc_msgpass(h, e1, src, dst)
    out = _node_mlp(eps1, h, p1, W11, b11, W21, b21)
    return out


# speedup vs baseline: 1.8437x; 1.8437x over previous
import functools

import jax
import jax.numpy as jnp
from jax import lax
from jax.experimental import pallas as pl
from jax.experimental.pallas import tpu as pltpu
from jax.experimental.pallas import tpu_sc as plsc

N = 10000
E = 320000
D = 128
DE = 16

N_PAD = 10112
NW = 32
E_PER_W = E // NW
CHUNK = 80
EPW_PAD = 10000
N_CHUNKS = EPW_PAD // CHUNK
E_IDX_PAD = NW * EPW_PAD


def _edge_mlp_body(ea_ref, we_ref, be_ref, out_ref):
    out_ref[...] = (
        jnp.dot(ea_ref[...], we_ref[...], preferred_element_type=jnp.float32)
        + be_ref[...]
    )


def _edge_mlp(edge_attr, We, be):
    blk = 8000
    grid = E // blk
    return pl.pallas_call(
        _edge_mlp_body,
        grid=(grid,),
        in_specs=[
            pl.BlockSpec((blk, DE), lambda i: (i, 0)),
            pl.BlockSpec((DE, D), lambda i: (0, 0)),
            pl.BlockSpec((1, D), lambda i: (0, 0)),
        ],
        out_specs=pl.BlockSpec((blk, D), lambda i: (i, 0)),
        out_shape=jax.ShapeDtypeStruct((E, D), jnp.float32),
    )(edge_attr, We, be.reshape(1, D))


def _sc_msgpass_body(x_hbm, e_hbm, src_hbm, dst_hbm, out_hbm,
                     sv0, sv1, dv0, dv1, xg0, xg1, ev0, ev1,
                     zbuf, aggr, isem0, isem1, gsem0, gsem1, esem0, esem1):
    c = lax.axis_index("c")
    s = lax.axis_index("s")
    wid = s * 2 + c
    ibase = wid * EPW_PAD
    ebase = wid * E_PER_W

    xg = (xg0, xg1)
    ev = (ev0, ev1)
    sv = (sv0, sv1)
    dv = (dv0, dv1)
    isem = (isem0, isem1)
    gsem = (gsem0, gsem1)
    esem = (esem0, esem1)

    def zrow(i, _):
        for j in range(D // 16):
            zbuf[i, pl.ds(j * 16, 16)] = jnp.zeros((16,), jnp.float32)
        return 0
    lax.fori_loop(0, 8, zrow, 0)

    rows_per_sub = N_PAD // 16
    def zcp(t, _):
        pltpu.sync_copy(zbuf, aggr.at[pl.ds(s * rows_per_sub + t * 8, 8)])
        return 0
    lax.fori_loop(0, rows_per_sub // 8, zcp, 0)

    plsc.subcore_barrier()

    def issue_idx(g, slot):
        pltpu.async_copy(src_hbm.at[pl.ds(ibase + g * CHUNK, CHUNK)],
                         sv[slot], isem[slot])
        pltpu.async_copy(dst_hbm.at[pl.ds(ibase + g * CHUNK, CHUNK)],
                         dv[slot], isem[slot])

    def wait_idx(slot):
        pltpu.make_async_copy(src_hbm.at[pl.ds(0, CHUNK)], sv[slot],
                              isem[slot]).wait()
        pltpu.make_async_copy(dst_hbm.at[pl.ds(0, CHUNK)], dv[slot],
                              isem[slot]).wait()

    def issue_data(g, slot):
        pltpu.async_copy(x_hbm.at[sv[slot]], xg[slot], gsem[slot])
        pltpu.async_copy(e_hbm.at[pl.ds(ebase + g * CHUNK, CHUNK)],
                         ev[slot], esem[slot])

    def wait_data(slot):
        pltpu.make_async_copy(x_hbm.at[sv[slot]], xg[slot], gsem[slot]).wait()
        pltpu.make_async_copy(e_hbm.at[pl.ds(0, CHUNK)], ev[slot],
                              esem[slot]).wait()

    def compute_scatter(slot):
        xgs, evs = xg[slot], ev[slot]
        @plsc.parallel_loop(0, CHUNK, unroll=2)
        def row(i):
            for j in range(D // 16):
                sl = pl.ds(j * 16, 16)
                xgs[i, sl] = jnp.maximum(xgs[i, sl] + evs[i, sl], 0.0)
        pltpu.sync_copy(xgs, aggr.at[dv[slot]], add=True)

    def half(g, p, q, prefetch_idx):
        wait_idx(q)
        issue_data(g + 1, q)
        wait_data(p)
        compute_scatter(p)
        if prefetch_idx:
            issue_idx(g + 2, p)

    issue_idx(0, 0)
    issue_idx(1, 1)
    wait_idx(0)
    issue_data(0, 0)

    def pair(t, _):
        half(2 * t, 0, 1, True)
        half(2 * t + 1, 1, 0, True)
        return 0
    lax.fori_loop(0, (N_CHUNKS - 3) // 2, pair, 0)
    half(N_CHUNKS - 3, 0, 1, True)
    half(N_CHUNKS - 2, 1, 0, False)
    wait_data(0)
    compute_scatter(0)

    plsc.subcore_barrier()

    def cpout(t, _):
        r = s * rows_per_sub + t * 64
        pltpu.sync_copy(aggr.at[pl.ds(r, 64)], out_hbm.at[c, pl.ds(r, 64)])
        return 0
    lax.fori_loop(0, rows_per_sub // 64, cpout, 0)
    r_tail = s * rows_per_sub + (rows_per_sub // 64) * 64
    pltpu.sync_copy(aggr.at[pl.ds(r_tail, rows_per_sub % 64)],
                    out_hbm.at[c, pl.ds(r_tail, rows_per_sub % 64)])


def _sc_msgpass(x, e, src_pad, dst_pad):
    mesh = plsc.VectorSubcoreMesh(core_axis_name="c", subcore_axis_name="s")
    f = pl.kernel(
        _sc_msgpass_body,
        out_type=jax.ShapeDtypeStruct((2, N_PAD, D), jnp.float32),
        mesh=mesh,
        scratch_types=[
            pltpu.VMEM((CHUNK,), jnp.int32),
            pltpu.VMEM((CHUNK,), jnp.int32),
            pltpu.VMEM((CHUNK,), jnp.int32),
            pltpu.VMEM((CHUNK,), jnp.int32),
            pltpu.VMEM((CHUNK, D), jnp.float32),
            pltpu.VMEM((CHUNK, D), jnp.float32),
            pltpu.VMEM((CHUNK, D), jnp.float32),
            pltpu.VMEM((CHUNK, D), jnp.float32),
            pltpu.VMEM((8, D), jnp.float32),
            pltpu.VMEM_SHARED((N_PAD, D), jnp.float32),
            pltpu.SemaphoreType.DMA,
            pltpu.SemaphoreType.DMA,
            pltpu.SemaphoreType.DMA,
            pltpu.SemaphoreType.DMA,
            pltpu.SemaphoreType.DMA,
            pltpu.SemaphoreType.DMA,
        ],
    )
    return f(x, e, src_pad, dst_pad)


def _node_mlp_body(eps_ref, x_ref, p_ref, w1_ref, b1_ref, w2_ref, b2_ref,
                   out_ref):
    p = p_ref[0] + p_ref[1]
    h = (1.0 + eps_ref[0, 0]) * x_ref[...] + p
    t = jnp.dot(h, w1_ref[...], preferred_element_type=jnp.float32) + b1_ref[...]
    t = 0.5 * t * (1.0 + lax.erf(t * 0.7071067811865476))
    out_ref[...] = (
        jnp.dot(t, w2_ref[...], preferred_element_type=jnp.float32) + b2_ref[...]
    )


def _node_mlp(eps, x, partial, W1, b1, W2, b2):
    blk = 2000
    grid = N // blk
    return pl.pallas_call(
        _node_mlp_body,
        grid=(grid,),
        in_specs=[
            pl.BlockSpec(memory_space=pltpu.SMEM),
            pl.BlockSpec((blk, D), lambda i: (i, 0)),
            pl.BlockSpec((2, blk, D), lambda i: (0, i, 0)),
            pl.BlockSpec((D, D), lambda i: (0, 0)),
            pl.BlockSpec((1, D), lambda i: (0, 0)),
            pl.BlockSpec((D, D), lambda i: (0, 0)),
            pl.BlockSpec((1, D), lambda i: (0, 0)),
        ],
        out_specs=pl.BlockSpec((blk, D), lambda i: (i, 0)),
        out_shape=jax.ShapeDtypeStruct((N, D), jnp.float32),
    )(eps.reshape(1, 1), x, partial, W1, b1.reshape(1, D), W2, b2.reshape(1, D))


def kernel(x, edge_attr, edge_idx, eps0, We0, be0, W10, b10, W20, b20,
           eps1, We1, be1, W11, b11, W21, b21):
    src = edge_idx[0]
    dst = edge_idx[1]

    e0 = _edge_mlp(edge_attr, We0, be0)
    e1 = _edge_mlp(edge_attr, We1, be1)

    p0 = _sc_msgpass(x, e0, src, dst)
    h = _node_mlp(eps0, x, p0, W10, b10, W20, b20)

    p1 = _sc_msgpass(h, e1, src, dst)
    out = _node_mlp(eps1, h, p1, W11, b11, W21, b21)
    return out


# speedup vs baseline: 1.8581x; 1.0078x over previous
import functools

import jax
import jax.numpy as jnp
from jax import lax
from jax.experimental import pallas as pl
from jax.experimental.pallas import tpu as pltpu
from jax.experimental.pallas import tpu_sc as plsc

N = 10000
E = 320000
D = 128
DE = 16

N_PAD = 10112
NW = 32
E_PER_W = E // NW
CHUNK = 80
EPW_PAD = 10000
N_CHUNKS = EPW_PAD // CHUNK
E_IDX_PAD = NW * EPW_PAD


def _edge_mlp_body(ea_ref, we_ref, be_ref, out_ref):
    out_ref[...] = (
        jnp.dot(ea_ref[...], we_ref[...], preferred_element_type=jnp.float32)
        + be_ref[...]
    )


def _edge_mlp(edge_attr, We, be):
    blk = 8000
    grid = E // blk
    return pl.pallas_call(
        _edge_mlp_body,
        grid=(grid,),
        in_specs=[
            pl.BlockSpec((blk, DE), lambda i: (i, 0)),
            pl.BlockSpec((DE, D), lambda i: (0, 0)),
            pl.BlockSpec((1, D), lambda i: (0, 0)),
        ],
        out_specs=pl.BlockSpec((blk, D), lambda i: (i, 0)),
        out_shape=jax.ShapeDtypeStruct((E, D), jnp.float32),
    )(edge_attr, We, be.reshape(1, D))


def _sc_msgpass_body(x_hbm, e_hbm, src_hbm, dst_hbm, out_hbm,
                     sv0, sv1, dv0, dv1, xg0, xg1, ev0, ev1,
                     zbuf, aggr, isem0, isem1, gsem0, gsem1, esem0, esem1):
    c = lax.axis_index("c")
    s = lax.axis_index("s")
    wid = s * 2 + c
    ibase = wid * EPW_PAD
    ebase = wid * E_PER_W

    xg = (xg0, xg1)
    ev = (ev0, ev1)
    sv = (sv0, sv1)
    dv = (dv0, dv1)
    isem = (isem0, isem1)
    gsem = (gsem0, gsem1)
    esem = (esem0, esem1)

    def zrow(i, _):
        for j in range(D // 16):
            zbuf[i, pl.ds(j * 16, 16)] = jnp.zeros((16,), jnp.float32)
        return 0
    lax.fori_loop(0, 8, zrow, 0)

    rows_per_sub = N_PAD // 16
    def zcp(t, _):
        pltpu.sync_copy(zbuf, aggr.at[pl.ds(s * rows_per_sub + t * 8, 8)])
        return 0
    lax.fori_loop(0, rows_per_sub // 8, zcp, 0)

    plsc.subcore_barrier()

    def issue_idx(g, slot):
        pltpu.async_copy(src_hbm.at[pl.ds(ibase + g * CHUNK, CHUNK)],
                         sv[slot], isem[slot])
        pltpu.async_copy(dst_hbm.at[pl.ds(ibase + g * CHUNK, CHUNK)],
                         dv[slot], isem[slot])

    def wait_idx(slot):
        pltpu.make_async_copy(src_hbm.at[pl.ds(0, CHUNK)], sv[slot],
                              isem[slot]).wait()
        pltpu.make_async_copy(dst_hbm.at[pl.ds(0, CHUNK)], dv[slot],
                              isem[slot]).wait()

    def issue_data(g, slot):
        pltpu.async_copy(x_hbm.at[sv[slot]], xg[slot], gsem[slot])
        pltpu.async_copy(e_hbm.at[pl.ds(ebase + g * CHUNK, CHUNK)],
                         ev[slot], esem[slot])

    def wait_data(slot):
        pltpu.make_async_copy(x_hbm.at[sv[slot]], xg[slot], gsem[slot]).wait()
        pltpu.make_async_copy(e_hbm.at[pl.ds(0, CHUNK)], ev[slot],
                              esem[slot]).wait()

    def compute_scatter(slot):
        xgs, evs = xg[slot], ev[slot]
        def row(i, _):
            for j in range(D // 16):
                sl = pl.ds(j * 16, 16)
                xgs[i, sl] = jnp.maximum(xgs[i, sl] + evs[i, sl], 0.0)
            return 0
        lax.fori_loop(0, CHUNK, row, 0)
        pltpu.sync_copy(xgs, aggr.at[dv[slot]], add=True)

    def half(g, p, q, prefetch_idx):
        wait_idx(q)
        issue_data(g + 1, q)
        wait_data(p)
        compute_scatter(p)
        if prefetch_idx:
            issue_idx(g + 2, p)

    issue_idx(0, 0)
    issue_idx(1, 1)
    wait_idx(0)
    issue_data(0, 0)

    def pair(t, _):
        half(2 * t, 0, 1, True)
        half(2 * t + 1, 1, 0, True)
        return 0
    lax.fori_loop(0, (N_CHUNKS - 3) // 2, pair, 0)
    half(N_CHUNKS - 3, 0, 1, True)
    half(N_CHUNKS - 2, 1, 0, False)
    wait_data(0)
    compute_scatter(0)

    plsc.subcore_barrier()

    def cpout(t, _):
        r = s * rows_per_sub + t * 64
        pltpu.sync_copy(aggr.at[pl.ds(r, 64)], out_hbm.at[c, pl.ds(r, 64)])
        return 0
    lax.fori_loop(0, rows_per_sub // 64, cpout, 0)
    r_tail = s * rows_per_sub + (rows_per_sub // 64) * 64
    pltpu.sync_copy(aggr.at[pl.ds(r_tail, rows_per_sub % 64)],
                    out_hbm.at[c, pl.ds(r_tail, rows_per_sub % 64)])


def _sc_msgpass(x, e, src_pad, dst_pad):
    mesh = plsc.VectorSubcoreMesh(core_axis_name="c", subcore_axis_name="s")
    f = pl.kernel(
        _sc_msgpass_body,
        out_type=jax.ShapeDtypeStruct((2, N_PAD, D), jnp.float32),
        mesh=mesh,
        scratch_types=[
            pltpu.VMEM((CHUNK,), jnp.int32),
            pltpu.VMEM((CHUNK,), jnp.int32),
            pltpu.VMEM((CHUNK,), jnp.int32),
            pltpu.VMEM((CHUNK,), jnp.int32),
            pltpu.VMEM((CHUNK, D), jnp.float32),
            pltpu.VMEM((CHUNK, D), jnp.float32),
            pltpu.VMEM((CHUNK, D), jnp.float32),
            pltpu.VMEM((CHUNK, D), jnp.float32),
            pltpu.VMEM((8, D), jnp.float32),
            pltpu.VMEM_SHARED((N_PAD, D), jnp.float32),
            pltpu.SemaphoreType.DMA,
            pltpu.SemaphoreType.DMA,
            pltpu.SemaphoreType.DMA,
            pltpu.SemaphoreType.DMA,
            pltpu.SemaphoreType.DMA,
            pltpu.SemaphoreType.DMA,
        ],
    )
    return f(x, e, src_pad, dst_pad)


def _node_mlp_body(eps_ref, x_ref, p_ref, w1_ref, b1_ref, w2_ref, b2_ref,
                   out_ref):
    p = p_ref[0] + p_ref[1]
    h = (1.0 + eps_ref[0, 0]) * x_ref[...] + p
    t = jnp.dot(h, w1_ref[...], preferred_element_type=jnp.float32) + b1_ref[...]
    t = 0.5 * t * (1.0 + lax.erf(t * 0.7071067811865476))
    out_ref[...] = (
        jnp.dot(t, w2_ref[...], preferred_element_type=jnp.float32) + b2_ref[...]
    )


def _node_mlp(eps, x, partial, W1, b1, W2, b2):
    blk = 2000
    grid = N // blk
    return pl.pallas_call(
        _node_mlp_body,
        grid=(grid,),
        in_specs=[
            pl.BlockSpec(memory_space=pltpu.SMEM),
            pl.BlockSpec((blk, D), lambda i: (i, 0)),
            pl.BlockSpec((2, blk, D), lambda i: (0, i, 0)),
            pl.BlockSpec((D, D), lambda i: (0, 0)),
            pl.BlockSpec((1, D), lambda i: (0, 0)),
            pl.BlockSpec((D, D), lambda i: (0, 0)),
            pl.BlockSpec((1, D), lambda i: (0, 0)),
        ],
        out_specs=pl.BlockSpec((blk, D), lambda i: (i, 0)),
        out_shape=jax.ShapeDtypeStruct((N, D), jnp.float32),
    )(eps.reshape(1, 1), x, partial, W1, b1.reshape(1, D), W2, b2.reshape(1, D))


def kernel(x, edge_attr, edge_idx, eps0, We0, be0, W10, b10, W20, b20,
           eps1, We1, be1, W11, b11, W21, b21):
    src = edge_idx[0]
    dst = edge_idx[1]

    e0 = _edge_mlp(edge_attr, We0, be0)
    e1 = _edge_mlp(edge_attr, We1, be1)

    p0 = _sc_msgpass(x, e0, src, dst)
    h = _node_mlp(eps0, x, p0, W10, b10, W20, b20)

    p1 = _sc_msgpass(h, e1, src, dst)
    out = _node_mlp(eps1, h, p1, W11, b11, W21, b21)
    return out
